# baseline (speedup 1.0000x reference)
import functools

import jax
import jax.numpy as jnp
from jax import lax
from jax.experimental import pallas as pl
from jax.experimental.pallas import tpu as pltpu
from jax.experimental.pallas import tpu_sc as plsc

_NUM_CLASSES = 1000000
_EMBED_DIM = 64
_BATCH = 16384
_HIST = 50
_HPAD = 56

_info = plsc.get_sparse_core_info()
_NC = _info.num_cores
_NS = _info.num_subcores
_NW = _NC * _NS
_IDX_PER_W = _BATCH * _HPAD // _NW
_CHUNK = 128
_NCHUNKS = _IDX_PER_W // _CHUNK
_GROUP = 2
_GCHUNK = _GROUP * _CHUNK
_NGROUPS = _NCHUNKS // _GROUP

_mesh = plsc.VectorSubcoreMesh(core_axis_name="c", subcore_axis_name="s")


@functools.partial(
    pl.kernel,
    out_type=jax.ShapeDtypeStruct((_BATCH * _HPAD, 128), jnp.float32),
    mesh=_mesh,
    scratch_types=[
        pltpu.VMEM((_NCHUNKS, _CHUNK), jnp.int32),
        pltpu.VMEM((2, _GCHUNK, 128), jnp.float32),
        pltpu.SemaphoreType.DMA,
        pltpu.SemaphoreType.DMA,
    ],
    compiler_params=pltpu.CompilerParams(use_tc_tiling_on_sc=False),
)
def _emb_lookup(x_hbm, staging_hbm, out_hbm, idx_v, rows_v, gsem, ssem):
    wid = lax.axis_index("s") * _NC + lax.axis_index("c")
    base = wid * _IDX_PER_W
    pltpu.sync_copy(x_hbm.at[wid], idx_v)

    def fire_gathers(g, p):
        for b in range(_GROUP):
            pltpu.async_copy(
                staging_hbm.at[idx_v.at[g * _GROUP + b]],
                rows_v.at[p, pl.ds(b * _CHUNK, _CHUNK)],
                gsem,
            )

    def wait_gathers(g, p):
        for b in range(_GROUP):
            pltpu.make_async_copy(
                staging_hbm.at[idx_v.at[g * _GROUP + b]],
                rows_v.at[p, pl.ds(b * _CHUNK, _CHUNK)],
                gsem,
            ).wait()

    def fire_store(g, p):
        pltpu.async_copy(
            rows_v.at[p], out_hbm.at[pl.ds(base + g * _GCHUNK, _GCHUNK)], ssem
        )

    def wait_store(g, p):
        pltpu.make_async_copy(
            rows_v.at[p], out_hbm.at[pl.ds(base + g * _GCHUNK, _GCHUNK)], ssem
        ).wait()

    fire_gathers(0, 0)
    wait_gathers(0, 0)
    fire_store(0, 0)
    fire_gathers(1, 1)

    @pl.loop(0, (_NGROUPS - 2) // 2)
    def _steady(t):
        g = 2 * t + 1
        wait_gathers(g, 1)
        fire_store(g, 1)
        wait_store(g - 1, 0)
        fire_gathers(g + 1, 0)
        wait_gathers(g + 1, 0)
        fire_store(g + 1, 0)
        wait_store(g, 1)
        fire_gathers(g + 2, 1)

    g_last = _NGROUPS - 1
    wait_gathers(g_last, 1)
    fire_store(g_last, 1)
    wait_store(g_last - 1, 0)
    wait_store(g_last, 1)


def kernel(x, table):
    staging = jnp.concatenate(
        [table, jnp.zeros((_NUM_CLASSES, 128 - _EMBED_DIM), jnp.float32)], axis=1
    )
    xpad = jnp.pad(x.astype(jnp.int32), ((0, 0), (0, _HPAD - _HIST)))
    xw = xpad.reshape(_NW, _NCHUNKS, _CHUNK)
    out2d = _emb_lookup(xw, staging)
    return out2d.reshape(_BATCH, _HPAD, 128)[:, :_HIST, :_EMBED_DIM]


# speedup vs baseline: 4.3003x; 4.3003x over previous
import functools

import jax
import jax.numpy as jnp
from jax import lax
from jax.experimental import pallas as pl
from jax.experimental.pallas import tpu as pltpu
from jax.experimental.pallas import tpu_sc as plsc

_NUM_CLASSES = 1000000
_EMBED_DIM = 64
_BATCH = 16384
_HIST = 50
_TOTAL = _BATCH * _HIST

_info = plsc.get_sparse_core_info()
_NC = _info.num_cores
_NS = _info.num_subcores
_NW = _NC * _NS
_B_PER_W = _TOTAL // _NW
_CHUNK = 128
_NCHUNKS = _B_PER_W // _CHUNK
_GROUP = 4
_GCHUNK = _GROUP * _CHUNK
_NGROUPS = _NCHUNKS // _GROUP

_mesh = plsc.VectorSubcoreMesh(core_axis_name="c", subcore_axis_name="s")


@functools.partial(
    pl.kernel,
    out_type=jax.ShapeDtypeStruct((_TOTAL, _EMBED_DIM), jnp.float32),
    mesh=_mesh,
    scratch_types=[
        pltpu.VMEM((_NCHUNKS, _CHUNK), jnp.int32),
        pltpu.VMEM((2, _GCHUNK, _EMBED_DIM), jnp.float32),
        pltpu.SemaphoreType.DMA,
        pltpu.SemaphoreType.DMA,
    ],
    compiler_params=pltpu.CompilerParams(
        use_tc_tiling_on_sc=False, skip_device_barrier=True
    ),
)
def _emb_lookup(idx_hbm, table_hbm, out_hbm, idx_v, rows_v, gsem, ssem):
    wid = lax.axis_index("s") * _NC + lax.axis_index("c")
    base = wid * _B_PER_W
    pltpu.sync_copy(idx_hbm.at[wid], idx_v)

    def fire_gathers(g, p):
        for b in range(_GROUP):
            pltpu.async_copy(
                table_hbm.at[idx_v.at[g * _GROUP + b]],
                rows_v.at[p, pl.ds(b * _CHUNK, _CHUNK)],
                gsem,
            )

    def wait_gathers(g, p):
        for b in range(_GROUP):
            pltpu.make_async_copy(
                table_hbm.at[idx_v.at[g * _GROUP + b]],
                rows_v.at[p, pl.ds(b * _CHUNK, _CHUNK)],
                gsem,
            ).wait()

    def fire_store(g, p):
        pltpu.async_copy(
            rows_v.at[p], out_hbm.at[pl.ds(base + g * _GCHUNK, _GCHUNK)], ssem
        )

    def wait_store(g, p):
        pltpu.make_async_copy(
            rows_v.at[p], out_hbm.at[pl.ds(base + g * _GCHUNK, _GCHUNK)], ssem
        ).wait()

    fire_gathers(0, 0)
    wait_gathers(0, 0)
    fire_store(0, 0)
    fire_gathers(1, 1)

    @pl.loop(0, (_NGROUPS - 2) // 2)
    def _steady(t):
        g = 2 * t + 1
        wait_gathers(g, 1)
        fire_store(g, 1)
        wait_store(g - 1, 0)
        fire_gathers(g + 1, 0)
        wait_gathers(g + 1, 0)
        fire_store(g + 1, 0)
        wait_store(g, 1)
        fire_gathers(g + 2, 1)

    g_last = _NGROUPS - 1
    wait_gathers(g_last, 1)
    fire_store(g_last, 1)
    wait_store(g_last - 1, 0)
    wait_store(g_last, 1)


def kernel(x, table):
    idx = x.reshape(_NW, _NCHUNKS, _CHUNK).astype(jnp.int32)
    out = _emb_lookup(idx, table)
    return out.reshape(_BATCH, _HIST, _EMBED_DIM)


# speedup vs baseline: 4.3299x; 1.0069x over previous
import functools

import jax
import jax.numpy as jnp
from jax import lax
from jax.experimental import pallas as pl
from jax.experimental.pallas import tpu as pltpu
from jax.experimental.pallas import tpu_sc as plsc

_NUM_CLASSES = 1000000
_EMBED_DIM = 64
_BATCH = 16384
_HIST = 50
_TOTAL = _BATCH * _HIST

_info = plsc.get_sparse_core_info()
_NC = _info.num_cores
_NS = _info.num_subcores
_NW = _NC * _NS
_B_PER_W = _TOTAL // _NW
_CHUNK = 128
_NCHUNKS = _B_PER_W // _CHUNK
_GROUP = 5
_GCHUNK = _GROUP * _CHUNK
_NGROUPS = _NCHUNKS // _GROUP

_mesh = plsc.VectorSubcoreMesh(core_axis_name="c", subcore_axis_name="s")


@functools.partial(
    pl.kernel,
    out_type=jax.ShapeDtypeStruct((_TOTAL, _EMBED_DIM), jnp.float32),
    mesh=_mesh,
    scratch_types=[
        pltpu.VMEM((_NCHUNKS, _CHUNK), jnp.int32),
        pltpu.VMEM((2, _GCHUNK, _EMBED_DIM), jnp.float32),
        pltpu.SemaphoreType.DMA,
        pltpu.SemaphoreType.DMA,
    ],
    compiler_params=pltpu.CompilerParams(use_tc_tiling_on_sc=False),
)
def _emb_lookup(idx_hbm, table_hbm, out_hbm, idx_v, rows_v, gsem, ssem):
    wid = lax.axis_index("s") * _NC + lax.axis_index("c")
    base = wid * _B_PER_W
    pltpu.sync_copy(idx_hbm.at[wid], idx_v)

    def fire_gathers(g, p):
        for b in range(_GROUP):
            pltpu.async_copy(
                table_hbm.at[idx_v.at[g * _GROUP + b]],
                rows_v.at[p, pl.ds(b * _CHUNK, _CHUNK)],
                gsem,
            )

    def wait_gathers(g, p):
        for b in range(_GROUP):
            pltpu.make_async_copy(
                table_hbm.at[idx_v.at[g * _GROUP + b]],
                rows_v.at[p, pl.ds(b * _CHUNK, _CHUNK)],
                gsem,
            ).wait()

    def fire_store(g, p):
        pltpu.async_copy(
            rows_v.at[p], out_hbm.at[pl.ds(base + g * _GCHUNK, _GCHUNK)], ssem
        )

    def wait_store(g, p):
        pltpu.make_async_copy(
            rows_v.at[p], out_hbm.at[pl.ds(base + g * _GCHUNK, _GCHUNK)], ssem
        ).wait()

    fire_gathers(0, 0)
    wait_gathers(0, 0)
    fire_store(0, 0)
    fire_gathers(1, 1)

    @pl.loop(0, (_NGROUPS - 2) // 2)
    def _steady(t):
        g = 2 * t + 1
        wait_gathers(g, 1)
        fire_store(g, 1)
        wait_store(g - 1, 0)
        fire_gathers(g + 1, 0)
        wait_gathers(g + 1, 0)
        fire_store(g + 1, 0)
        wait_store(g, 1)
        fire_gathers(g + 2, 1)

    g_last = _NGROUPS - 1
    wait_gathers(g_last, 1)
    fire_store(g_last, 1)
    wait_store(g_last - 1, 0)
    wait_store(g_last, 1)


def kernel(x, table):
    idx = x.reshape(_NW, _NCHUNKS, _CHUNK).astype(jnp.int32)
    out = _emb_lookup(idx, table)
    return out.reshape(_BATCH, _HIST, _EMBED_DIM)
